# baseline (speedup 1.0000x reference)
import functools

import jax
import jax.numpy as jnp
from jax import lax
from jax.experimental import pallas as pl
from jax.experimental.pallas import tpu as pltpu
from jax.experimental.pallas import tpu_sc as plsc

NUM_ITEMS = 100000
N = NUM_ITEMS + 1
NW = 16
LSUB = 16
SUBW = 392
CHUNK = LSUB * SUBW
NPAD = NW * CHUNK
TOK = 1024 * 200
TOKW = TOK // NW
Q = 1024 * 100
QW = Q // (NW * 2)
B = 1024
NVB = B // LSUB
NR = 32
SUBR = CHUNK // NR
HALF = LSUB * SUBR
HB = LSUB * B

_mesh = plsc.VectorSubcoreMesh(
    core_axis_name="c", subcore_axis_name="s", num_cores=2)


@functools.partial(
    pl.kernel,
    out_type=jax.ShapeDtypeStruct((Q,), jnp.float32),
    mesh=_mesh,
    compiler_params=pltpu.CompilerParams(needs_layout_passes=False),
    scratch_types=dict(
        countsT=pltpu.VMEM_SHARED((NPAD,), jnp.int32),
        wtab=pltpu.VMEM_SHARED((NW * B,), jnp.int32),
        recipT=pltpu.VMEM_SHARED((NPAD,), jnp.float32),
        tokbuf=pltpu.VMEM((TOKW,), jnp.int32),
        onesb=pltpu.VMEM((TOKW,), jnp.int32),
        counts_c=pltpu.VMEM((CHUNK,), jnp.int32),
        tie_c=pltpu.VMEM((CHUNK,), jnp.int32),
        recip_c=pltpu.VMEM((CHUNK,), jnp.float32),
        hist2d=pltpu.VMEM((NR * B,), jnp.int32),
        tmpB=pltpu.VMEM((B,), jnp.int32),
        psw=pltpu.VMEM((B,), jnp.int32),
        htot=pltpu.VMEM((B,), jnp.int32),
        glb=pltpu.VMEM((B,), jnp.int32),
        wall=pltpu.VMEM((NW * B,), jnp.int32),
        qidx=pltpu.VMEM((QW,), jnp.int32),
        qout=pltpu.VMEM((QW,), jnp.float32),
        gsem=pltpu.SemaphoreType.DMA,
        tsem=pltpu.SemaphoreType.DMA,
    ),
)
def _pop_kernel(tok_hbm, q_hbm, out_hbm, *, countsT, wtab, recipT, tokbuf,
                onesb, counts_c, tie_c, recip_c, hist2d, tmpB, psw, htot,
                glb, wall, qidx, qout, gsem, tsem):
    w = lax.axis_index("s")
    qw = lax.axis_index("c") * NW + w
    lane = jnp.arange(LSUB, dtype=jnp.int32)
    zeros16 = jnp.zeros((LSUB,), jnp.int32)
    ones16 = jnp.ones((LSUB,), jnp.int32)
    lane_mul = lane * SUBW
    lane_B = lane * B

    tok_dma = pltpu.async_copy(tok_hbm.at[pl.ds(w * TOKW, TOKW)], tokbuf,
                               tsem)
    q_dma = pltpu.async_copy(q_hbm.at[pl.ds(qw * QW, QW)], qidx, gsem)

    def _fill(ref, n, val):
        @plsc.parallel_loop(0, n // LSUB, unroll=8)
        def body(v):
            ref[pl.ds(v * LSUB, LSUB)] = val

    _fill(hist2d, NR * B, zeros16)
    _fill(onesb, TOKW, ones16)
    pltpu.sync_copy(hist2d.at[pl.ds(0, CHUNK)],
                    countsT.at[pl.ds(w * CHUNK, CHUNK)])
    plsc.subcore_barrier()

    tok_dma.wait()
    pltpu.sync_copy(onesb, countsT.at[tokbuf], add=True)
    plsc.subcore_barrier()

    pltpu.sync_copy(countsT.at[pl.ds(w * CHUNK, CHUNK)], counts_c)

    lane_r = lane * SUBR
    c0a = plsc.load_gather(counts_c, [lane_r])
    c0b = plsc.load_gather(counts_c, [lane_r + HALF])

    def tie_body(t, carry):
        ca, cb, mx = carry
        tn = jnp.minimum(t + 1, SUBR - 1)
        ca_n = plsc.load_gather(counts_c, [lane_r + tn])
        cb_n = plsc.load_gather(counts_c, [lane_r + (HALF + tn)])
        cla = jnp.minimum(ca, B - 1)
        clb = jnp.minimum(cb, B - 1)
        ha = lane_B + cla
        hb = lane_B + (HB + clb)
        ta = plsc.load_gather(hist2d, [ha])
        tb = plsc.load_gather(hist2d, [hb])
        plsc.store_scatter(tie_c, [lane_r + t], ta * B + cla)
        plsc.store_scatter(tie_c, [lane_r + (HALF + t)], tb * B + clb)
        plsc.addupdate_scatter(hist2d, [ha], ones16, mask=ca < B)
        plsc.addupdate_scatter(hist2d, [hb], ones16, mask=cb < B)
        return ca_n, cb_n, jnp.maximum(mx, jnp.maximum(ca, cb))
    _, _, maxv = lax.fori_loop(0, SUBR, tie_body, (c0a, c0b, zeros16))
    maxc = jnp.max(maxv)

    @plsc.parallel_loop(0, NVB, unroll=2)
    def wsum_body(v):
        s = zeros16
        for l in range(NR):
            sl = hist2d[pl.ds(l * B + v * LSUB, LSUB)]
            hist2d[pl.ds(l * B + v * LSUB, LSUB)] = s
            s = s + sl
        tmpB[pl.ds(v * LSUB, LSUB)] = s
    pltpu.sync_copy(tmpB, wtab.at[pl.ds(w * B, B)])
    plsc.subcore_barrier()

    pltpu.sync_copy(wtab, wall)

    @plsc.parallel_loop(0, NVB, unroll=2)
    def wpre_body(v):
        sl = pl.ds(v * LSUB, LSUB)
        ht = zeros16
        ps = zeros16
        for j in range(NW):
            row = wall[pl.ds(j * B + v * LSUB, LSUB)]
            ht = ht + row
            ps = ps + jnp.where(j < w, row, 0)
        htot[sl] = ht
        psw[sl] = ps

    @plsc.parallel_loop(0, NVB, carry=jnp.int32(0))
    def g_body(v, carry):
        sl = pl.ds(v * LSUB, LSUB)
        vals = htot[sl]
        glb[sl] = (NPAD + 1) - (plsc.cumsum(vals) + carry) + psw[sl]
        return carry + jnp.sum(vals)

    @plsc.parallel_loop(0, SUBR, unroll=4)
    def rank_body(t):
        for half, hoff in ((0, 0), (HALF, HB)):
            idxs = lane_r + (half + t)
            combo = plsc.load_gather(tie_c, [idxs])
            cl = combo & (B - 1)
            tie = lax.shift_right_logical(combo, 10)
            g = plsc.load_gather(glb, [cl])
            ps2 = plsc.load_gather(hist2d, [lane_B + (hoff + cl)])
            rank = g + ps2 + tie
            plsc.store_scatter(recip_c, [idxs],
                               1.0 / rank.astype(jnp.float32))

    @pl.when(maxc >= B)
    def _heavy_fixup():
        def t_body(t, _):
            idxs = lane_mul + t
            c = plsc.load_gather(counts_c, [idxs])
            nh = jnp.sum((c >= B).astype(jnp.int32))

            @pl.when(nh > 0)
            def _():
                def k_body(k, _):
                    ck = jnp.sum(jnp.where(lane == k, c, 0))

                    @pl.when(ck >= B)
                    def _():
                        gi = w * CHUNK + k * SUBW + t

                        def outer(sw, acc):
                            pltpu.sync_copy(
                                countsT.at[pl.ds(sw * CHUNK, CHUNK)], tie_c)

                            def inner(v, a):
                                cv = tie_c[pl.ds(v * LSUB, LSUB)]
                                gidx = sw * CHUNK + v * LSUB + lane
                                m_gt = cv > ck
                                m_tie = (cv == ck) & (gidx < gi)
                                return (a + jnp.sum(m_gt.astype(jnp.int32))
                                        + jnp.sum(m_tie.astype(jnp.int32)))
                            return lax.fori_loop(0, SUBW, inner, acc)
                        nge = lax.fori_loop(0, NW, outer, jnp.int32(0))
                        rank = (1 + nge).astype(jnp.float32)
                        pos = k * SUBW + t
                        plsc.store_scatter(recip_c, [lane * 0 + pos],
                                           jnp.full((LSUB,), 1.0,
                                                    jnp.float32) / rank,
                                           mask=lane == 0)
                    return 0
                lax.fori_loop(0, LSUB, k_body, 0)
            return 0
        lax.fori_loop(0, SUBW, t_body, 0)

    pltpu.sync_copy(recip_c, recipT.at[pl.ds(w * CHUNK, CHUNK)])
    plsc.subcore_barrier()

    q_dma.wait()
    pltpu.async_copy(recipT.at[qidx], qout, gsem).wait()
    pltpu.sync_copy(qout, out_hbm.at[pl.ds(qw * QW, QW)])


@jax.jit
def kernel(input_seqs, poss_item_seqs):
    scores = _pop_kernel(input_seqs.reshape(-1), poss_item_seqs.reshape(-1))
    return scores.reshape(poss_item_seqs.shape)


# speedup vs baseline: 1.0117x; 1.0117x over previous
import functools

import jax
import jax.numpy as jnp
from jax import lax
from jax.experimental import pallas as pl
from jax.experimental.pallas import tpu as pltpu
from jax.experimental.pallas import tpu_sc as plsc

NUM_ITEMS = 100000
N = NUM_ITEMS + 1
NW = 16
LSUB = 16
SUBW = 392
CHUNK = LSUB * SUBW
NPAD = NW * CHUNK
TOK = 1024 * 200
TOKW = TOK // NW
Q = 1024 * 100
QW = Q // NW
B = 1024
NVB = B // LSUB
NR = 32
SUBR = CHUNK // NR
HALF = LSUB * SUBR
HB = LSUB * B

_mesh = plsc.VectorSubcoreMesh(
    core_axis_name="c", subcore_axis_name="s", num_cores=1)


@functools.partial(
    pl.kernel,
    out_type=jax.ShapeDtypeStruct((Q,), jnp.float32),
    mesh=_mesh,
    compiler_params=pltpu.CompilerParams(needs_layout_passes=False),
    scratch_types=dict(
        countsT=pltpu.VMEM_SHARED((NPAD,), jnp.int32),
        wtab=pltpu.VMEM_SHARED((NW * B,), jnp.int32),
        recipT=pltpu.VMEM_SHARED((NPAD,), jnp.float32),
        tokbuf=pltpu.VMEM((TOKW,), jnp.int32),
        onesb=pltpu.VMEM((TOKW,), jnp.int32),
        counts_c=pltpu.VMEM((CHUNK,), jnp.int32),
        tie_c=pltpu.VMEM((CHUNK,), jnp.int32),
        recip_c=pltpu.VMEM((CHUNK,), jnp.float32),
        hist2d=pltpu.VMEM((NR * B,), jnp.int32),
        tmpB=pltpu.VMEM((B,), jnp.int32),
        psw=pltpu.VMEM((B,), jnp.int32),
        htot=pltpu.VMEM((B,), jnp.int32),
        glb=pltpu.VMEM((B,), jnp.int32),
        wall=pltpu.VMEM((NW * B,), jnp.int32),
        qidx=pltpu.VMEM((QW,), jnp.int32),
        qout=pltpu.VMEM((QW,), jnp.float32),
        gsem=pltpu.SemaphoreType.DMA,
        tsem=pltpu.SemaphoreType.DMA,
    ),
)
def _pop_kernel(tok_hbm, q_hbm, out_hbm, *, countsT, wtab, recipT, tokbuf,
                onesb, counts_c, tie_c, recip_c, hist2d, tmpB, psw, htot,
                glb, wall, qidx, qout, gsem, tsem):
    w = lax.axis_index("s")
    lane = jnp.arange(LSUB, dtype=jnp.int32)
    zeros16 = jnp.zeros((LSUB,), jnp.int32)
    ones16 = jnp.ones((LSUB,), jnp.int32)
    lane_mul = lane * SUBW
    lane_B = lane * B

    tok_dma = pltpu.async_copy(tok_hbm.at[pl.ds(w * TOKW, TOKW)], tokbuf,
                               tsem)
    q_dma = pltpu.async_copy(q_hbm.at[pl.ds(w * QW, QW)], qidx, gsem)

    def _fill(ref, n, val):
        @plsc.parallel_loop(0, n // LSUB, unroll=8)
        def body(v):
            ref[pl.ds(v * LSUB, LSUB)] = val

    _fill(hist2d, NR * B, zeros16)
    _fill(onesb, TOKW, ones16)
    pltpu.sync_copy(hist2d.at[pl.ds(0, CHUNK)],
                    countsT.at[pl.ds(w * CHUNK, CHUNK)])
    plsc.subcore_barrier()

    tok_dma.wait()
    pltpu.sync_copy(onesb, countsT.at[tokbuf], add=True)
    plsc.subcore_barrier()

    pltpu.sync_copy(countsT.at[pl.ds(w * CHUNK, CHUNK)], counts_c)

    lane_r = lane * SUBR
    c0a = plsc.load_gather(counts_c, [lane_r])
    c0b = plsc.load_gather(counts_c, [lane_r + HALF])

    def tie_body(t, carry):
        ca, cb, mx = carry
        tn = jnp.minimum(t + 1, SUBR - 1)
        ca_n = plsc.load_gather(counts_c, [lane_r + tn])
        cb_n = plsc.load_gather(counts_c, [lane_r + (HALF + tn)])
        cla = jnp.minimum(ca, B - 1)
        clb = jnp.minimum(cb, B - 1)
        ha = lane_B + cla
        hb = lane_B + (HB + clb)
        ta = plsc.load_gather(hist2d, [ha])
        tb = plsc.load_gather(hist2d, [hb])
        plsc.store_scatter(tie_c, [lane_r + t], ta * B + cla)
        plsc.store_scatter(tie_c, [lane_r + (HALF + t)], tb * B + clb)
        plsc.addupdate_scatter(hist2d, [ha], ones16, mask=ca < B)
        plsc.addupdate_scatter(hist2d, [hb], ones16, mask=cb < B)
        return ca_n, cb_n, jnp.maximum(mx, jnp.maximum(ca, cb))
    _, _, maxv = lax.fori_loop(0, SUBR, tie_body, (c0a, c0b, zeros16))
    maxc = jnp.max(maxv)

    @plsc.parallel_loop(0, NVB, unroll=2)
    def wsum_body(v):
        s = zeros16
        for l in range(NR):
            sl = hist2d[pl.ds(l * B + v * LSUB, LSUB)]
            hist2d[pl.ds(l * B + v * LSUB, LSUB)] = s
            s = s + sl
        tmpB[pl.ds(v * LSUB, LSUB)] = s
    pltpu.sync_copy(tmpB, wtab.at[pl.ds(w * B, B)])
    plsc.subcore_barrier()

    pltpu.sync_copy(wtab, wall)

    @plsc.parallel_loop(0, NVB, unroll=2)
    def wpre_body(v):
        sl = pl.ds(v * LSUB, LSUB)
        ht = zeros16
        ps = zeros16
        for j in range(NW):
            row = wall[pl.ds(j * B + v * LSUB, LSUB)]
            ht = ht + row
            ps = ps + jnp.where(j < w, row, 0)
        htot[sl] = ht
        psw[sl] = ps

    @plsc.parallel_loop(0, NVB, carry=jnp.int32(0))
    def g_body(v, carry):
        sl = pl.ds(v * LSUB, LSUB)
        vals = htot[sl]
        glb[sl] = (NPAD + 1) - (plsc.cumsum(vals) + carry) + psw[sl]
        return carry + jnp.sum(vals)

    @plsc.parallel_loop(0, SUBR, unroll=4)
    def rank_body(t):
        for half, hoff in ((0, 0), (HALF, HB)):
            idxs = lane_r + (half + t)
            combo = plsc.load_gather(tie_c, [idxs])
            cl = combo & (B - 1)
            tie = lax.shift_right_logical(combo, 10)
            g = plsc.load_gather(glb, [cl])
            ps2 = plsc.load_gather(hist2d, [lane_B + (hoff + cl)])
            rank = g + ps2 + tie
            plsc.store_scatter(recip_c, [idxs],
                               1.0 / rank.astype(jnp.float32))

    @pl.when(maxc >= B)
    def _heavy_fixup():
        def t_body(t, _):
            idxs = lane_mul + t
            c = plsc.load_gather(counts_c, [idxs])
            nh = jnp.sum((c >= B).astype(jnp.int32))

            @pl.when(nh > 0)
            def _():
                def k_body(k, _):
                    ck = jnp.sum(jnp.where(lane == k, c, 0))

                    @pl.when(ck >= B)
                    def _():
                        gi = w * CHUNK + k * SUBW + t

                        def outer(sw, acc):
                            pltpu.sync_copy(
                                countsT.at[pl.ds(sw * CHUNK, CHUNK)], tie_c)

                            def inner(v, a):
                                cv = tie_c[pl.ds(v * LSUB, LSUB)]
                                gidx = sw * CHUNK + v * LSUB + lane
                                m_gt = cv > ck
                                m_tie = (cv == ck) & (gidx < gi)
                                return (a + jnp.sum(m_gt.astype(jnp.int32))
                                        + jnp.sum(m_tie.astype(jnp.int32)))
                            return lax.fori_loop(0, SUBW, inner, acc)
                        nge = lax.fori_loop(0, NW, outer, jnp.int32(0))
                        rank = (1 + nge).astype(jnp.float32)
                        pos = k * SUBW + t
                        plsc.store_scatter(recip_c, [lane * 0 + pos],
                                           jnp.full((LSUB,), 1.0,
                                                    jnp.float32) / rank,
                                           mask=lane == 0)
                    return 0
                lax.fori_loop(0, LSUB, k_body, 0)
            return 0
        lax.fori_loop(0, SUBW, t_body, 0)

    pltpu.sync_copy(recip_c, recipT.at[pl.ds(w * CHUNK, CHUNK)])
    plsc.subcore_barrier()

    q_dma.wait()
    h = QW // 2
    g1 = pltpu.async_copy(recipT.at[qidx.at[pl.ds(0, h)]],
                          qout.at[pl.ds(0, h)], gsem)
    g2 = pltpu.async_copy(recipT.at[qidx.at[pl.ds(h, h)]],
                          qout.at[pl.ds(h, h)], tsem)
    g1.wait()
    g2.wait()
    pltpu.sync_copy(qout, out_hbm.at[pl.ds(w * QW, QW)])


@jax.jit
def kernel(input_seqs, poss_item_seqs):
    scores = _pop_kernel(input_seqs.reshape(-1), poss_item_seqs.reshape(-1))
    return scores.reshape(poss_item_seqs.shape)
